# baseline (speedup 1.0000x reference)
import functools

import jax
import jax.numpy as jnp
from jax import lax
from jax.experimental import pallas as pl
from jax.experimental.pallas import tpu as pltpu
from jax.experimental.pallas import tpu_sc as plsc

_B = 8
_C = 256
_E = 20000
_K = 5000
_L = 16
_KP = 5120
_NC = 2
_NS = 16
_MPC = _B // _NC
_TPM = _NS // _MPC
_CPT = _C // _TPM
_CB = 64
_NCB = _C // _CB
_NV = _E // _L
_RADIX = 256
_NPASS = 4


def _keys_body(ec_ref, x_ref, keys_ref):
    b = pl.program_id(0)
    xb = x_ref[0]
    u = None
    for j in range(16):
        a = xb[8 * j:8 * j + 8, :]
        c = xb[128 + 8 * j:136 + 8 * j, :]
        tj = a * a + c * c
        u = tj if u is None else u + tj
    w = u[0:4, :] + u[4:8, :]
    y = w[0:2, :] + w[2:4, :]
    s = y[0:1, :] + y[1:2, :]
    ub = lax.bitcast_convert_type(s, jnp.uint32)
    neg = ub >= jnp.uint32(0x80000000)
    mono = jnp.where(neg, ~ub, ub | jnp.uint32(0x80000000))
    key = ~mono
    eidx = lax.broadcasted_iota(jnp.int32, (1, _E), 1)
    key = jnp.where(eidx < ec_ref[b], key, jnp.uint32(0xFFFFFFFF))
    keys_ref[...] = lax.bitcast_convert_type(key, jnp.int32).reshape(1, 1, _E)


_keys_call = pl.pallas_call(
    _keys_body,
    grid=(_B,),
    in_specs=[
        pl.BlockSpec(memory_space=pltpu.MemorySpace.SMEM),
        pl.BlockSpec((1, _C, _E), lambda b: (b, 0, 0)),
    ],
    out_specs=pl.BlockSpec((1, 1, _E), lambda b: (b, 0, 0)),
    out_shape=jax.ShapeDtypeStruct((_B, 1, _E), jnp.int32),
)


def _sc_body(keys_hbm, x_hbm, out_hbm,
             ka, kb, va, vb, hist, bins, ids, row, outb, ids_shared):
    cid = lax.axis_index("c")
    sid = lax.axis_index("s")
    lane = lax.iota(jnp.int32, _L)

    @pl.when(sid < _MPC)
    def _sort():
        m = cid * _MPC + sid
        pltpu.sync_copy(keys_hbm.at[pl.ds(m * _E, _E)], ka)

        def init_body(i, _):
            va[pl.ds(i * _L, _L)] = i * _L + lane
            return 0
        lax.fori_loop(0, _NV, init_body, 0)

        bufs = [(ka, va, kb, vb), (kb, vb, ka, va)]
        for p in range(_NPASS):
            sk, sv, dk, dv = bufs[p % 2]
            shift = 8 * p
            zero16 = jnp.zeros((_L,), jnp.int32)
            ones16 = jnp.ones((_L,), jnp.int32)

            def z_body(i, _):
                hist[pl.ds(i * _L, _L)] = zero16
                return 0
            lax.fori_loop(0, _RADIX, z_body, 0)

            def h_body(i, _):
                k = sk[pl.ds(i * _L, _L)]
                d = lax.shift_right_logical(k, shift) & 0xFF
                hi = d * _L + lane
                plsc.store_scatter(hist, [hi],
                                   plsc.load_gather(hist, [hi]) + ones16)
                return 0
            lax.fori_loop(0, _NV, h_body, 0)

            def b_body(g, carry):
                dg = g * _L + lane
                tot = jnp.zeros((_L,), jnp.int32)
                for l in range(_L):
                    tot = tot + plsc.load_gather(hist, [dg * _L + l])
                incl = plsc.cumsum(tot)
                bins[pl.ds(g * _L, _L)] = incl - tot + carry
                return carry + jnp.sum(tot)
            lax.fori_loop(0, _RADIX // _L, b_body, jnp.int32(0))

            def p_body(i, _):
                k = sk[pl.ds(i * _L, _L)]
                v = sv[pl.ds(i * _L, _L)]
                d = lax.shift_right_logical(k, shift) & 0xFF
                base = plsc.load_gather(bins, [d])
                cnt, last = plsc.scan_count(d)
                pos = base + cnt - 1
                plsc.store_scatter(dk, [pos], k)
                plsc.store_scatter(dv, [pos], v)
                plsc.store_scatter(bins, [d], pos + 1, mask=last)
                return 0
            lax.fori_loop(0, _NV, p_body, 0)

        pltpu.sync_copy(va.at[pl.ds(0, _KP)], ids_shared.at[sid])

    plsc.subcore_barrier()

    ml = sid // _TPM
    pltpu.sync_copy(ids_shared.at[ml], ids)
    row0 = (cid * _MPC + ml) * _C + (sid % _TPM) * _CPT

    def r_body(r, _):
        rg = row0 + r
        pltpu.sync_copy(x_hbm.at[pl.ds(rg * _E, _E)], row)

        def g_body(j, _):
            idx = ids[pl.ds(j * _L, _L)]
            outb[pl.ds(j * _L, _L)] = plsc.load_gather(row, [idx])
            return 0
        lax.fori_loop(0, _KP // _L, g_body, 0)
        pltpu.sync_copy(outb.at[pl.ds(0, _K)], out_hbm.at[pl.ds(rg * _K, _K)])
        return 0
    lax.fori_loop(0, _CPT, r_body, 0)


@functools.cache
def _get_sc_call():
    return functools.partial(
        pl.kernel,
        out_type=jax.ShapeDtypeStruct((_B * _C * _K,), jnp.float32),
        mesh=plsc.VectorSubcoreMesh(
            core_axis_name="c", subcore_axis_name="s",
            num_cores=_NC, num_subcores=_NS),
        compiler_params=pltpu.CompilerParams(needs_layout_passes=False),
        scratch_types=[
            pltpu.VMEM((_E,), jnp.int32),
            pltpu.VMEM((_E,), jnp.int32),
            pltpu.VMEM((_E,), jnp.int32),
            pltpu.VMEM((_E,), jnp.int32),
            pltpu.VMEM((_RADIX * _L,), jnp.int32),
            pltpu.VMEM((_RADIX,), jnp.int32),
            pltpu.VMEM((_KP,), jnp.int32),
            pltpu.VMEM((_E,), jnp.float32),
            pltpu.VMEM((_KP,), jnp.float32),
            pltpu.VMEM_SHARED((_MPC, _KP), jnp.int32),
        ],
    )(_sc_body)


def kernel(x, edges_count):
    keys = _keys_call(edges_count, x).reshape(_B * _E)
    out = _get_sc_call()(keys, x.reshape(_B * _C * _E))
    return out.reshape(_B, _C, _K)


# speedup vs baseline: 1.0413x; 1.0413x over previous
import functools

import jax
import jax.numpy as jnp
from jax import lax
from jax.experimental import pallas as pl
from jax.experimental.pallas import tpu as pltpu
from jax.experimental.pallas import tpu_sc as plsc

_B = 8
_C = 256
_E = 20000
_K = 5000
_L = 16
_KP = 5120
_KO = 5008
_NC = 2
_NS = 16
_MPC = _B // _NC
_TPM = _NS // _MPC
_CPT = _C // _TPM
_CB = 64
_NCB = _C // _CB
_NV = _E // _L
_RADIX = 256
_NPASS = 4


def _keys_body(ec_ref, x_ref, keys_ref):
    b = pl.program_id(0)
    xb = x_ref[0]
    u = None
    for j in range(16):
        a = xb[8 * j:8 * j + 8, :]
        c = xb[128 + 8 * j:136 + 8 * j, :]
        tj = a * a + c * c
        u = tj if u is None else u + tj
    w = u[0:4, :] + u[4:8, :]
    y = w[0:2, :] + w[2:4, :]
    s = y[0:1, :] + y[1:2, :]
    ub = lax.bitcast_convert_type(s, jnp.uint32)
    neg = ub >= jnp.uint32(0x80000000)
    mono = jnp.where(neg, ~ub, ub | jnp.uint32(0x80000000))
    key = ~mono
    eidx = lax.broadcasted_iota(jnp.int32, (1, _E), 1)
    key = jnp.where(eidx < ec_ref[b], key, jnp.uint32(0xFFFFFFFF))
    keys_ref[...] = lax.bitcast_convert_type(key, jnp.int32).reshape(1, 1, _E)


_keys_call = pl.pallas_call(
    _keys_body,
    grid=(_B,),
    in_specs=[
        pl.BlockSpec(memory_space=pltpu.MemorySpace.SMEM),
        pl.BlockSpec((1, _C, _E), lambda b: (b, 0, 0)),
    ],
    out_specs=pl.BlockSpec((1, 1, _E), lambda b: (b, 0, 0)),
    out_shape=jax.ShapeDtypeStruct((_B, 1, _E), jnp.int32),
)


def _sc_body(keys_hbm, x_hbm, out_hbm, ids_hbm,
             ka, kb, va, vb, hist, bins, row, rowb, outb,
             sem_a, sem_b):
    cid = lax.axis_index("c")
    sid = lax.axis_index("s")
    lane = lax.iota(jnp.int32, _L)

    @pl.when(sid < _MPC)
    def _sort():
        m = cid * _MPC + sid
        pltpu.sync_copy(keys_hbm.at[pl.ds(m * _E, _E)], ka)

        def init_body(i, _):
            va[pl.ds(i * _L, _L)] = i * _L + lane
            return 0
        lax.fori_loop(0, _NV, init_body, 0)

        bufs = [(ka, va, kb, vb), (kb, vb, ka, va)]
        for p in range(_NPASS):
            sk, sv, dk, dv = bufs[p % 2]
            shift = 8 * p
            zero16 = jnp.zeros((_L,), jnp.int32)
            ones16 = jnp.ones((_L,), jnp.int32)

            def z_body(i, _):
                hist[pl.ds(i * _L, _L)] = zero16
                return 0
            lax.fori_loop(0, _RADIX, z_body, 0)

            def h_body(i, _):
                k = sk[pl.ds(i * _L, _L)]
                d = lax.shift_right_logical(k, shift) & 0xFF
                hi = d * _L + lane
                plsc.store_scatter(hist, [hi],
                                   plsc.load_gather(hist, [hi]) + ones16)
                return 0
            lax.fori_loop(0, _NV, h_body, 0)

            def b_body(g, carry):
                dg = g * _L + lane
                tot = jnp.zeros((_L,), jnp.int32)
                for l in range(_L):
                    tot = tot + plsc.load_gather(hist, [dg * _L + l])
                incl = plsc.cumsum(tot)
                bins[pl.ds(g * _L, _L)] = incl - tot + carry
                return carry + jnp.sum(tot)
            lax.fori_loop(0, _RADIX // _L, b_body, jnp.int32(0))

            def p_body(i, _):
                k = sk[pl.ds(i * _L, _L)]
                v = sv[pl.ds(i * _L, _L)]
                d = lax.shift_right_logical(k, shift) & 0xFF
                base = plsc.load_gather(bins, [d])
                cnt, last = plsc.scan_count(d)
                pos = base + cnt - 1
                plsc.store_scatter(dk, [pos], k)
                plsc.store_scatter(dv, [pos], v)
                plsc.store_scatter(bins, [d], pos + 1, mask=last)
                return 0
            lax.fori_loop(0, _NV, p_body, 0)

        pltpu.sync_copy(va.at[pl.ds(0, _KP)],
                        ids_hbm.at[pl.ds(m * _KP, _KP)])

    plsc.subcore_barrier()

    ml = sid // _TPM
    pltpu.sync_copy(ids_hbm.at[pl.ds((cid * _MPC + ml) * _KP, _KP)],
                    vb.at[pl.ds(0, _KP)])
    row0 = (cid * _MPC + ml) * _C + (sid % _TPM) * _CPT

    bufs = [(row, sem_a), (rowb, sem_b)]

    def _fetch(r, buf, sem):
        return pltpu.async_copy(x_hbm.at[pl.ds((row0 + r) * _E, _E)], buf, sem)

    desc = [_fetch(0, *bufs[0]), None]
    for r in range(_CPT):
        buf, _ = bufs[r % 2]
        if r + 1 < _CPT:
            desc[(r + 1) % 2] = _fetch(r + 1, *bufs[(r + 1) % 2])
        desc[r % 2].wait()

        def g_body(j, _):
            idx = vb[pl.ds(j * _L, _L)]
            outb[pl.ds(j * _L, _L)] = plsc.load_gather(buf, [idx])
            return 0
        lax.fori_loop(0, _KO // _L, g_body, 0, unroll=8)
        pltpu.sync_copy(outb.at[pl.ds(0, _K)],
                        out_hbm.at[pl.ds((row0 + r) * _K, _K)])


@functools.cache
def _get_sc_call():
    return functools.partial(
        pl.kernel,
        out_type=(jax.ShapeDtypeStruct((_B * _C * _K,), jnp.float32),
                  jax.ShapeDtypeStruct((_B * _KP,), jnp.int32)),
        mesh=plsc.VectorSubcoreMesh(
            core_axis_name="c", subcore_axis_name="s",
            num_cores=_NC, num_subcores=_NS),
        compiler_params=pltpu.CompilerParams(needs_layout_passes=False),
        scratch_types=[
            pltpu.VMEM((_E,), jnp.int32),
            pltpu.VMEM((_E,), jnp.int32),
            pltpu.VMEM((_E,), jnp.int32),
            pltpu.VMEM((_E,), jnp.int32),
            pltpu.VMEM((_RADIX * _L,), jnp.int32),
            pltpu.VMEM((_RADIX,), jnp.int32),
            pltpu.VMEM((_E,), jnp.float32),
            pltpu.VMEM((_E,), jnp.float32),
            pltpu.VMEM((_KO,), jnp.float32),
            pltpu.SemaphoreType.DMA,
            pltpu.SemaphoreType.DMA,
        ],
    )(_sc_body)


def kernel(x, edges_count):
    keys = _keys_call(edges_count, x).reshape(_B * _E)
    out, _ = _get_sc_call()(keys, x.reshape(_B * _C * _E))
    return out.reshape(_B, _C, _K)


# speedup vs baseline: 1.0530x; 1.0112x over previous
import functools

import jax
import jax.numpy as jnp
from jax import lax
from jax.experimental import pallas as pl
from jax.experimental.pallas import tpu as pltpu
from jax.experimental.pallas import tpu_sc as plsc

_B = 8
_C = 256
_E = 20000
_K = 5000
_L = 16
_KP = 5120
_KO = 5008
_NC = 2
_NS = 16
_MPC = _B // _NC
_TPM = _NS // _MPC
_CPT = _C // _TPM
_CB = 64
_NCB = _C // _CB
_NV = _E // _L
_RADIX = 256
_NPASS = 4


def _keys_body(ec_ref, x_ref, keys_ref):
    b = pl.program_id(0)
    xb = x_ref[0]
    u = None
    for j in range(16):
        a = xb[8 * j:8 * j + 8, :]
        c = xb[128 + 8 * j:136 + 8 * j, :]
        tj = a * a + c * c
        u = tj if u is None else u + tj
    w = u[0:4, :] + u[4:8, :]
    y = w[0:2, :] + w[2:4, :]
    s = y[0:1, :] + y[1:2, :]
    ub = lax.bitcast_convert_type(s, jnp.uint32)
    neg = ub >= jnp.uint32(0x80000000)
    mono = jnp.where(neg, ~ub, ub | jnp.uint32(0x80000000))
    key = ~mono
    eidx = lax.broadcasted_iota(jnp.int32, (1, _E), 1)
    key = jnp.where(eidx < ec_ref[b], key, jnp.uint32(0xFFFFFFFF))
    keys_ref[...] = lax.bitcast_convert_type(key, jnp.int32).reshape(1, 1, _E)


_keys_call = pl.pallas_call(
    _keys_body,
    grid=(_B,),
    in_specs=[
        pl.BlockSpec(memory_space=pltpu.MemorySpace.SMEM),
        pl.BlockSpec((1, _C, _E), lambda b: (b, 0, 0)),
    ],
    out_specs=pl.BlockSpec((1, 1, _E), lambda b: (b, 0, 0)),
    out_shape=jax.ShapeDtypeStruct((_B, 1, _E), jnp.int32),
    compiler_params=pltpu.CompilerParams(vmem_limit_bytes=100 * 1024 * 1024),
)


def _sc_body(keys_hbm, x_hbm, out_hbm, ids_hbm,
             ka, kb, va, vb, hist, bins, row, rowb, outb,
             sem_a, sem_b):
    cid = lax.axis_index("c")
    sid = lax.axis_index("s")
    lane = lax.iota(jnp.int32, _L)

    @pl.when(sid < _MPC)
    def _sort():
        m = cid * _MPC + sid
        pltpu.sync_copy(keys_hbm.at[pl.ds(m * _E, _E)], ka)

        def init_body(i, _):
            va[pl.ds(i * _L, _L)] = i * _L + lane
            return 0
        lax.fori_loop(0, _NV, init_body, 0)

        bufs = [(ka, va, kb, vb), (kb, vb, ka, va)]
        for p in range(_NPASS):
            sk, sv, dk, dv = bufs[p % 2]
            shift = 8 * p
            zero16 = jnp.zeros((_L,), jnp.int32)
            ones16 = jnp.ones((_L,), jnp.int32)

            def z_body(i, _):
                hist[pl.ds(i * _L, _L)] = zero16
                return 0
            lax.fori_loop(0, _RADIX, z_body, 0)

            def h_body(i, _):
                k = sk[pl.ds(i * _L, _L)]
                d = lax.shift_right_logical(k, shift) & 0xFF
                hi = d * _L + lane
                plsc.store_scatter(hist, [hi],
                                   plsc.load_gather(hist, [hi]) + ones16)
                return 0
            lax.fori_loop(0, _NV, h_body, 0, unroll=4)

            def b_body(g, carry):
                dg = g * _L + lane
                tot = jnp.zeros((_L,), jnp.int32)
                for l in range(_L):
                    tot = tot + plsc.load_gather(hist, [dg * _L + l])
                incl = plsc.cumsum(tot)
                bins[pl.ds(g * _L, _L)] = incl - tot + carry
                return carry + jnp.sum(tot)
            lax.fori_loop(0, _RADIX // _L, b_body, jnp.int32(0))

            def p_body(i, _):
                k = sk[pl.ds(i * _L, _L)]
                v = sv[pl.ds(i * _L, _L)]
                d = lax.shift_right_logical(k, shift) & 0xFF
                base = plsc.load_gather(bins, [d])
                cnt, last = plsc.scan_count(d)
                pos = base + cnt - 1
                plsc.store_scatter(dk, [pos], k)
                plsc.store_scatter(dv, [pos], v)
                plsc.store_scatter(bins, [d], pos + 1, mask=last)
                return 0
            lax.fori_loop(0, _NV, p_body, 0, unroll=2)

        pltpu.sync_copy(va.at[pl.ds(0, _KP)],
                        ids_hbm.at[pl.ds(m * _KP, _KP)])

    plsc.subcore_barrier()

    ml = sid // _TPM
    pltpu.sync_copy(ids_hbm.at[pl.ds((cid * _MPC + ml) * _KP, _KP)],
                    vb.at[pl.ds(0, _KP)])
    row0 = (cid * _MPC + ml) * _C + (sid % _TPM) * _CPT

    bufs = [(row, sem_a), (rowb, sem_b)]

    def _fetch(r, buf, sem):
        return pltpu.async_copy(x_hbm.at[pl.ds((row0 + r) * _E, _E)], buf, sem)

    desc = [_fetch(0, *bufs[0]), None]
    for r in range(_CPT):
        buf, _ = bufs[r % 2]
        if r + 1 < _CPT:
            desc[(r + 1) % 2] = _fetch(r + 1, *bufs[(r + 1) % 2])
        desc[r % 2].wait()

        def g_body(j, _):
            idx = vb[pl.ds(j * _L, _L)]
            outb[pl.ds(j * _L, _L)] = plsc.load_gather(buf, [idx])
            return 0
        lax.fori_loop(0, _KO // _L, g_body, 0, unroll=8)
        pltpu.sync_copy(outb.at[pl.ds(0, _K)],
                        out_hbm.at[pl.ds((row0 + r) * _K, _K)])


@functools.cache
def _get_sc_call():
    return functools.partial(
        pl.kernel,
        out_type=(jax.ShapeDtypeStruct((_B * _C * _K,), jnp.float32),
                  jax.ShapeDtypeStruct((_B * _KP,), jnp.int32)),
        mesh=plsc.VectorSubcoreMesh(
            core_axis_name="c", subcore_axis_name="s",
            num_cores=_NC, num_subcores=_NS),
        compiler_params=pltpu.CompilerParams(needs_layout_passes=False),
        scratch_types=[
            pltpu.VMEM((_E,), jnp.int32),
            pltpu.VMEM((_E,), jnp.int32),
            pltpu.VMEM((_E,), jnp.int32),
            pltpu.VMEM((_E,), jnp.int32),
            pltpu.VMEM((_RADIX * _L,), jnp.int32),
            pltpu.VMEM((_RADIX,), jnp.int32),
            pltpu.VMEM((_E,), jnp.float32),
            pltpu.VMEM((_E,), jnp.float32),
            pltpu.VMEM((_KO,), jnp.float32),
            pltpu.SemaphoreType.DMA,
            pltpu.SemaphoreType.DMA,
        ],
    )(_sc_body)


def kernel(x, edges_count):
    keys = _keys_call(edges_count, x).reshape(_B * _E)
    out, _ = _get_sc_call()(keys, x.reshape(_B * _C * _E))
    return out.reshape(_B, _C, _K)
